# baseline (speedup 1.0000x reference)
import functools

import jax
import jax.numpy as jnp
from jax import lax
from jax.experimental import pallas as pl
from jax.experimental.pallas import tpu as pltpu
from jax.experimental.pallas import tpu_sc as plsc

E = 320000
N = 10000
D = 128

NC = 2
NS = 16
NW = NC * NS
RPW = E // NW
CH = 80
NCHUNK = RPW // CH

UNIT = NW * CH
UNITS = (30, 30, 30, 29, 6)
CK = len(UNITS)

NIO = 10
RIO = N // NIO
ZB = 40

_SC_MESH = plsc.VectorSubcoreMesh(
    core_axis_name="c", subcore_axis_name="s", num_cores=NC, num_subcores=NS)


def _proj_body(nf_ref, ws_ref, wd_ref, ns_ref, nd_ref):
    x = nf_ref[...]
    ns_ref[...] = jnp.dot(x, ws_ref[...], preferred_element_type=jnp.float32)
    nd_ref[...] = jnp.dot(x, wd_ref[...], preferred_element_type=jnp.float32)


def _project(nfeat, w1s, w1d):
    blk = 2000
    return pl.pallas_call(
        _proj_body,
        out_shape=(jax.ShapeDtypeStruct((N, D), jnp.float32),
                   jax.ShapeDtypeStruct((N, D), jnp.float32)),
        grid=(N // blk,),
        in_specs=[pl.BlockSpec((blk, D), lambda i: (i, 0)),
                  pl.BlockSpec((D, D), lambda i: (0, 0)),
                  pl.BlockSpec((D, D), lambda i: (0, 0))],
        out_specs=(pl.BlockSpec((blk, D), lambda i: (i, 0)),
                   pl.BlockSpec((blk, D), lambda i: (i, 0))),
    )(nfeat, w1s, w1d)


def _make_gather_body(nchc):
    def _gather_body(sd_hbm, ns_hbm, nd_hbm, out_hbm,
                     sdidx, bufa, sem_a, sem_b, sem_o):
        c = lax.axis_index("c")
        s = lax.axis_index("s")
        wid = s * NC + c
        base = wid * (nchc * CH)
        pltpu.sync_copy(sd_hbm.at[wid], sdidx)

        pltpu.async_copy(ns_hbm.at[sdidx.at[0, 0]], bufa.at[0], sem_a.at[0])

        @pl.loop(0, nchc)
        def chunk(j):
            slot = lax.rem(j, 2)
            nslot = 1 - slot

            pltpu.make_async_copy(ns_hbm.at[sdidx.at[j, 0]], bufa.at[slot],
                                  sem_a.at[slot]).wait()
            pltpu.async_copy(nd_hbm.at[sdidx.at[j, 1]], bufa.at[slot],
                             sem_b.at[slot], add=True)

            @pl.when(j + 1 < nchc)
            def _issue_next():
                @pl.when(j >= 1)
                def _drain_out():
                    pltpu.make_async_copy(
                        bufa.at[nslot],
                        out_hbm.at[pl.ds(base + (j - 1) * CH, CH)],
                        sem_o.at[nslot]).wait()

                pltpu.async_copy(ns_hbm.at[sdidx.at[j + 1, 0]], bufa.at[nslot],
                                 sem_a.at[nslot])

            pltpu.make_async_copy(nd_hbm.at[sdidx.at[j, 1]], bufa.at[slot],
                                  sem_b.at[slot]).wait()
            pltpu.async_copy(bufa.at[slot],
                             out_hbm.at[pl.ds(base + j * CH, CH)],
                             sem_o.at[slot])

        for tail in (nchc - 2, nchc - 1):
            slot = tail % 2
            pltpu.make_async_copy(
                bufa.at[slot], out_hbm.at[pl.ds(base + tail * CH, CH)],
                sem_o.at[slot]).wait()

    return _gather_body


@functools.cache
def _gather_u(nchc):
    return pl.kernel(
        _make_gather_body(nchc),
        out_type=jax.ShapeDtypeStruct((nchc * UNIT, D), jnp.float32),
        mesh=_SC_MESH,
        scratch_types=[
            pltpu.VMEM((nchc, 2, CH), jnp.int32),
            pltpu.VMEM((2, CH, D), jnp.float32),
            pltpu.SemaphoreType.DMA((2,)),
            pltpu.SemaphoreType.DMA((2,)),
            pltpu.SemaphoreType.DMA((2,)),
        ],
    )


def _edge_body(_buf_ref, ef_ref, g_ref, w1_ref, w2_ref, b1_ref, b2_ref,
               gg_ref, bt_ref, out_ref):
    x = ef_ref[...]
    pre = (jnp.dot(x, w1_ref[...], preferred_element_type=jnp.float32)
           + g_ref[...] + b1_ref[...])
    h = pre * jax.nn.sigmoid(pre)
    y = jnp.dot(h, w2_ref[...], preferred_element_type=jnp.float32) + b2_ref[...]
    m = jnp.mean(y, axis=-1, keepdims=True)
    d = y - m
    v = jnp.mean(d * d, axis=-1, keepdims=True)
    out_ref[...] = d * lax.rsqrt(v + 1e-5) * gg_ref[...] + bt_ref[...] + x


_EBLK = UNIT


def _edge_body0(ef_ref, g_ref, w1_ref, w2_ref, b1_ref, b2_ref, gg_ref, bt_ref,
                out_ref):
    _edge_body(None, ef_ref, g_ref, w1_ref, w2_ref, b1_ref, b2_ref, gg_ref,
               bt_ref, out_ref)


def _edge_mlp_chunk(k0, u, buf, efeat, g_k, w1e, eW2, eb1, eb2, eg, ebt):
    vec = lambda: pl.BlockSpec((1, D), lambda i: (0, 0))
    row_specs = [
        pl.BlockSpec((_EBLK, D), lambda i, k0=k0: (k0 + i, 0)),
        pl.BlockSpec((_EBLK, D), lambda i: (i, 0)),
        pl.BlockSpec((D, D), lambda i: (0, 0)),
        pl.BlockSpec((D, D), lambda i: (0, 0)),
        vec(), vec(), vec(), vec(),
    ]
    args = (efeat, g_k, w1e, eW2, eb1, eb2, eg, ebt)
    if k0 == 0:
        return pl.pallas_call(
            _edge_body0,
            out_shape=jax.ShapeDtypeStruct((E, D), jnp.float32),
            grid=(u,),
            in_specs=row_specs,
            out_specs=pl.BlockSpec((_EBLK, D), lambda i: (i, 0)),
        )(*args)
    return pl.pallas_call(
        _edge_body,
        out_shape=jax.ShapeDtypeStruct((E, D), jnp.float32),
        grid=(u,),
        in_specs=[pl.BlockSpec(memory_space=pl.ANY)] + row_specs,
        out_specs=pl.BlockSpec((_EBLK, D), lambda i, k0=k0: (k0 + i, 0)),
        input_output_aliases={0: 0},
    )(buf, *args)


def _scatter_body(ef_hbm, dst_hbm, out_hbm, didx, rows, agg, sem_r):
    c = lax.axis_index("c")
    s = lax.axis_index("s")
    wid = s * NC + c
    base = wid * RPW

    @pl.when(s < NIO)
    def _init():
        r0ref = rows.at[0]

        @pl.loop(0, ZB)
        def zrow(r):
            for cc in range(8):
                r0ref[r, pl.ds(cc * 16, 16)] = jnp.zeros((16,), jnp.float32)

        for k in range(RIO // ZB):
            pltpu.sync_copy(rows.at[0, pl.ds(0, ZB)],
                            agg.at[pl.ds(s * RIO + k * ZB, ZB)])

    plsc.subcore_barrier()

    pltpu.sync_copy(dst_hbm.at[wid], didx)
    pltpu.async_copy(ef_hbm.at[pl.ds(base, CH)], rows.at[0], sem_r.at[0])

    @pl.loop(0, NCHUNK)
    def chunk(j):
        slot = lax.rem(j, 2)
        nslot = 1 - slot

        @pl.when(j + 1 < NCHUNK)
        def _issue_next():
            pltpu.async_copy(ef_hbm.at[pl.ds(base + (j + 1) * CH, CH)],
                             rows.at[nslot], sem_r.at[nslot])

        pltpu.make_async_copy(ef_hbm.at[pl.ds(base + j * CH, CH)],
                              rows.at[slot], sem_r.at[slot]).wait()
        pltpu.sync_copy(rows.at[slot], agg.at[didx.at[j]], add=True)

    plsc.subcore_barrier()

    @pl.when(s < NIO)
    def _writeout():
        for k in range(RIO // ZB):
            r0 = s * RIO + k * ZB
            pltpu.sync_copy(agg.at[pl.ds(r0, ZB)], rows.at[0, pl.ds(0, ZB)])
            pltpu.sync_copy(rows.at[0, pl.ds(0, ZB)],
                            out_hbm.at[c, pl.ds(r0, ZB)])


_scatter = pl.kernel(
    _scatter_body,
    out_type=jax.ShapeDtypeStruct((NC, N, D), jnp.float32),
    mesh=_SC_MESH,
    scratch_types=[
        pltpu.VMEM((NCHUNK, CH), jnp.int32),
        pltpu.VMEM((2, CH, D), jnp.float32),
        pltpu.VMEM_SHARED((N, D), jnp.float32),
        pltpu.SemaphoreType.DMA((2,)),
    ],
)


def _node_body(p0_ref, p1_ref, nf_ref, w1a_ref, w1n_ref, w2_ref, b1_ref,
               b2_ref, gg_ref, bt_ref, out_ref):
    a = p0_ref[...] + p1_ref[...]
    x = nf_ref[...]
    pre = (jnp.dot(a, w1a_ref[...], preferred_element_type=jnp.float32)
           + jnp.dot(x, w1n_ref[...], preferred_element_type=jnp.float32)
           + b1_ref[...])
    h = pre * jax.nn.sigmoid(pre)
    y = jnp.dot(h, w2_ref[...], preferred_element_type=jnp.float32) + b2_ref[...]
    m = jnp.mean(y, axis=-1, keepdims=True)
    d = y - m
    v = jnp.mean(d * d, axis=-1, keepdims=True)
    out_ref[...] = d * lax.rsqrt(v + 1e-5) * gg_ref[...] + bt_ref[...] + x


def _node_mlp(p0, p1, nfeat, w1a, w1n, nW2, nb1, nb2, ng, nbt):
    blk = 2000
    vec = lambda: pl.BlockSpec((1, D), lambda i: (0, 0))
    mat = lambda: pl.BlockSpec((D, D), lambda i: (0, 0))
    return pl.pallas_call(
        _node_body,
        out_shape=jax.ShapeDtypeStruct((N, D), jnp.float32),
        grid=(N // blk,),
        in_specs=[pl.BlockSpec((blk, D), lambda i: (i, 0)),
                  pl.BlockSpec((blk, D), lambda i: (i, 0)),
                  pl.BlockSpec((blk, D), lambda i: (i, 0)),
                  mat(), mat(), mat(),
                  vec(), vec(), vec(), vec()],
        out_specs=pl.BlockSpec((blk, D), lambda i: (i, 0)),
    )(p0, p1, nfeat, w1a, w1n, nW2, nb1, nb2, ng, nbt)


def kernel(efeat, nfeat, edge_index, eW1, eb1, eW2, eb2, eg, ebt,
           nW1, nb1, nW2, nb2, ng, nbt):
    src = edge_index[0]
    dst = edge_index[1]
    dst3 = dst.reshape(NW, NCHUNK, CH)
    w1e, w1s, w1d = eW1[0:D], eW1[D:2 * D], eW1[2 * D:3 * D]

    ns, nd = _project(nfeat, w1s, w1d)

    gs = []
    off = 0
    for u in UNITS:
        sz = u * UNIT
        sd = jnp.stack([src[off:off + sz].reshape(NW, u, CH),
                        dst[off:off + sz].reshape(NW, u, CH)], axis=2)
        gs.append(_gather_u(u)(sd, ns, nd))
        off += sz

    buf = None
    k0 = 0
    for u, g_k in zip(UNITS, gs):
        buf = _edge_mlp_chunk(k0, u, buf, efeat, g_k, w1e, eW2,
                              eb1.reshape(1, D), eb2.reshape(1, D),
                              eg.reshape(1, D), ebt.reshape(1, D))
        k0 += u
    efeat_new = buf
    aggp = _scatter(efeat_new, dst3)
    nfeat_new = _node_mlp(aggp[0], aggp[1], nfeat, nW1[0:D], nW1[D:2 * D],
                          nW2, nb1.reshape(1, D), nb2.reshape(1, D),
                          ng.reshape(1, D), nbt.reshape(1, D))
    return (efeat_new, nfeat_new)


# speedup vs baseline: 1.0144x; 1.0144x over previous
import functools

import jax
import jax.numpy as jnp
from jax import lax
from jax.experimental import pallas as pl
from jax.experimental.pallas import tpu as pltpu
from jax.experimental.pallas import tpu_sc as plsc

E = 320000
N = 10000
D = 128

NC = 2
NS = 16
NW = NC * NS
RPW = E // NW
CH = 80
NCHUNK = RPW // CH

UNIT = NW * CH
UNITS = (30, 30, 30, 29, 6)
CK = len(UNITS)
NBUF = 3

NIO = 10
RIO = N // NIO
ZB = 40

_SC_MESH = plsc.VectorSubcoreMesh(
    core_axis_name="c", subcore_axis_name="s", num_cores=NC, num_subcores=NS)


def _proj_body(nf_ref, ws_ref, wd_ref, ns_ref, nd_ref):
    x = nf_ref[...]
    ns_ref[...] = jnp.dot(x, ws_ref[...], preferred_element_type=jnp.float32)
    nd_ref[...] = jnp.dot(x, wd_ref[...], preferred_element_type=jnp.float32)


def _project(nfeat, w1s, w1d):
    blk = 2000
    return pl.pallas_call(
        _proj_body,
        out_shape=(jax.ShapeDtypeStruct((N, D), jnp.float32),
                   jax.ShapeDtypeStruct((N, D), jnp.float32)),
        grid=(N // blk,),
        in_specs=[pl.BlockSpec((blk, D), lambda i: (i, 0)),
                  pl.BlockSpec((D, D), lambda i: (0, 0)),
                  pl.BlockSpec((D, D), lambda i: (0, 0))],
        out_specs=(pl.BlockSpec((blk, D), lambda i: (i, 0)),
                   pl.BlockSpec((blk, D), lambda i: (i, 0))),
    )(nfeat, w1s, w1d)


def _make_gather_body(nchc):
    def _gather_body(sd_hbm, ns_hbm, nd_hbm, out_hbm,
                     sdidx, bufa, sem_a, sem_b, sem_o):
        c = lax.axis_index("c")
        s = lax.axis_index("s")
        wid = s * NC + c
        base = wid * (nchc * CH)
        pltpu.sync_copy(sd_hbm.at[wid], sdidx)

        pltpu.async_copy(ns_hbm.at[sdidx.at[0, 0]], bufa.at[0], sem_a.at[0])
        pltpu.async_copy(ns_hbm.at[sdidx.at[1, 0]], bufa.at[1], sem_a.at[1])

        @pl.loop(0, nchc)
        def chunk(j):
            slot = lax.rem(j, NBUF)

            pltpu.make_async_copy(ns_hbm.at[sdidx.at[j, 0]], bufa.at[slot],
                                  sem_a.at[slot]).wait()
            pltpu.async_copy(nd_hbm.at[sdidx.at[j, 1]], bufa.at[slot],
                             sem_b.at[slot], add=True)

            @pl.when(j + 2 < nchc)
            def _issue_next():
                nslot = lax.rem(j + 2, NBUF)

                @pl.when(j >= 1)
                def _drain_out():
                    pltpu.make_async_copy(
                        bufa.at[nslot],
                        out_hbm.at[pl.ds(base + (j - 1) * CH, CH)],
                        sem_o.at[nslot]).wait()

                pltpu.async_copy(ns_hbm.at[sdidx.at[j + 2, 0]],
                                 bufa.at[nslot], sem_a.at[nslot])

            pltpu.make_async_copy(nd_hbm.at[sdidx.at[j, 1]], bufa.at[slot],
                                  sem_b.at[slot]).wait()
            pltpu.async_copy(bufa.at[slot],
                             out_hbm.at[pl.ds(base + j * CH, CH)],
                             sem_o.at[slot])

        for tail in (nchc - 3, nchc - 2, nchc - 1):
            slot = tail % NBUF
            pltpu.make_async_copy(
                bufa.at[slot], out_hbm.at[pl.ds(base + tail * CH, CH)],
                sem_o.at[slot]).wait()

    return _gather_body


@functools.cache
def _gather_u(nchc):
    return pl.kernel(
        _make_gather_body(nchc),
        out_type=jax.ShapeDtypeStruct((nchc * UNIT, D), jnp.float32),
        mesh=_SC_MESH,
        scratch_types=[
            pltpu.VMEM((nchc, 2, CH), jnp.int32),
            pltpu.VMEM((NBUF, CH, D), jnp.float32),
            pltpu.SemaphoreType.DMA((NBUF,)),
            pltpu.SemaphoreType.DMA((NBUF,)),
            pltpu.SemaphoreType.DMA((NBUF,)),
        ],
    )


def _edge_body(_buf_ref, ef_ref, g_ref, w1_ref, w2_ref, b1_ref, b2_ref,
               gg_ref, bt_ref, out_ref):
    x = ef_ref[...]
    pre = (jnp.dot(x, w1_ref[...], preferred_element_type=jnp.float32)
           + g_ref[...] + b1_ref[...])
    h = pre * jax.nn.sigmoid(pre)
    y = jnp.dot(h, w2_ref[...], preferred_element_type=jnp.float32) + b2_ref[...]
    m = jnp.mean(y, axis=-1, keepdims=True)
    d = y - m
    v = jnp.mean(d * d, axis=-1, keepdims=True)
    out_ref[...] = d * lax.rsqrt(v + 1e-5) * gg_ref[...] + bt_ref[...] + x


_EBLK = UNIT


def _edge_body0(ef_ref, g_ref, w1_ref, w2_ref, b1_ref, b2_ref, gg_ref, bt_ref,
                out_ref):
    _edge_body(None, ef_ref, g_ref, w1_ref, w2_ref, b1_ref, b2_ref, gg_ref,
               bt_ref, out_ref)


def _edge_mlp_chunk(k0, u, buf, efeat, g_k, w1e, eW2, eb1, eb2, eg, ebt):
    vec = lambda: pl.BlockSpec((1, D), lambda i: (0, 0))
    row_specs = [
        pl.BlockSpec((_EBLK, D), lambda i, k0=k0: (k0 + i, 0)),
        pl.BlockSpec((_EBLK, D), lambda i: (i, 0)),
        pl.BlockSpec((D, D), lambda i: (0, 0)),
        pl.BlockSpec((D, D), lambda i: (0, 0)),
        vec(), vec(), vec(), vec(),
    ]
    args = (efeat, g_k, w1e, eW2, eb1, eb2, eg, ebt)
    if k0 == 0:
        return pl.pallas_call(
            _edge_body0,
            out_shape=jax.ShapeDtypeStruct((E, D), jnp.float32),
            grid=(u,),
            in_specs=row_specs,
            out_specs=pl.BlockSpec((_EBLK, D), lambda i: (i, 0)),
        )(*args)
    return pl.pallas_call(
        _edge_body,
        out_shape=jax.ShapeDtypeStruct((E, D), jnp.float32),
        grid=(u,),
        in_specs=[pl.BlockSpec(memory_space=pl.ANY)] + row_specs,
        out_specs=pl.BlockSpec((_EBLK, D), lambda i, k0=k0: (k0 + i, 0)),
        input_output_aliases={0: 0},
    )(buf, *args)


def _scatter_body(ef_hbm, dst_hbm, out_hbm, didx, rows, agg, sem_r):
    c = lax.axis_index("c")
    s = lax.axis_index("s")
    wid = s * NC + c
    base = wid * RPW

    @pl.when(s < NIO)
    def _init():
        r0ref = rows.at[0]

        @pl.loop(0, ZB)
        def zrow(r):
            for cc in range(8):
                r0ref[r, pl.ds(cc * 16, 16)] = jnp.zeros((16,), jnp.float32)

        for k in range(RIO // ZB):
            pltpu.sync_copy(rows.at[0, pl.ds(0, ZB)],
                            agg.at[pl.ds(s * RIO + k * ZB, ZB)])

    plsc.subcore_barrier()

    pltpu.sync_copy(dst_hbm.at[wid], didx)
    pltpu.async_copy(ef_hbm.at[pl.ds(base, CH)], rows.at[0], sem_r.at[0])

    @pl.loop(0, NCHUNK)
    def chunk(j):
        slot = lax.rem(j, 2)
        nslot = 1 - slot

        @pl.when(j + 1 < NCHUNK)
        def _issue_next():
            pltpu.async_copy(ef_hbm.at[pl.ds(base + (j + 1) * CH, CH)],
                             rows.at[nslot], sem_r.at[nslot])

        pltpu.make_async_copy(ef_hbm.at[pl.ds(base + j * CH, CH)],
                              rows.at[slot], sem_r.at[slot]).wait()
        pltpu.sync_copy(rows.at[slot], agg.at[didx.at[j]], add=True)

    plsc.subcore_barrier()

    @pl.when(s < NIO)
    def _writeout():
        for k in range(RIO // ZB):
            r0 = s * RIO + k * ZB
            pltpu.sync_copy(agg.at[pl.ds(r0, ZB)], rows.at[0, pl.ds(0, ZB)])
            pltpu.sync_copy(rows.at[0, pl.ds(0, ZB)],
                            out_hbm.at[c, pl.ds(r0, ZB)])


_scatter = pl.kernel(
    _scatter_body,
    out_type=jax.ShapeDtypeStruct((NC, N, D), jnp.float32),
    mesh=_SC_MESH,
    scratch_types=[
        pltpu.VMEM((NCHUNK, CH), jnp.int32),
        pltpu.VMEM((2, CH, D), jnp.float32),
        pltpu.VMEM_SHARED((N, D), jnp.float32),
        pltpu.SemaphoreType.DMA((2,)),
    ],
)


def _node_body(p0_ref, p1_ref, nf_ref, w1a_ref, w1n_ref, w2_ref, b1_ref,
               b2_ref, gg_ref, bt_ref, out_ref):
    a = p0_ref[...] + p1_ref[...]
    x = nf_ref[...]
    pre = (jnp.dot(a, w1a_ref[...], preferred_element_type=jnp.float32)
           + jnp.dot(x, w1n_ref[...], preferred_element_type=jnp.float32)
           + b1_ref[...])
    h = pre * jax.nn.sigmoid(pre)
    y = jnp.dot(h, w2_ref[...], preferred_element_type=jnp.float32) + b2_ref[...]
    m = jnp.mean(y, axis=-1, keepdims=True)
    d = y - m
    v = jnp.mean(d * d, axis=-1, keepdims=True)
    out_ref[...] = d * lax.rsqrt(v + 1e-5) * gg_ref[...] + bt_ref[...] + x


def _node_mlp(p0, p1, nfeat, w1a, w1n, nW2, nb1, nb2, ng, nbt):
    blk = 2000
    vec = lambda: pl.BlockSpec((1, D), lambda i: (0, 0))
    mat = lambda: pl.BlockSpec((D, D), lambda i: (0, 0))
    return pl.pallas_call(
        _node_body,
        out_shape=jax.ShapeDtypeStruct((N, D), jnp.float32),
        grid=(N // blk,),
        in_specs=[pl.BlockSpec((blk, D), lambda i: (i, 0)),
                  pl.BlockSpec((blk, D), lambda i: (i, 0)),
                  pl.BlockSpec((blk, D), lambda i: (i, 0)),
                  mat(), mat(), mat(),
                  vec(), vec(), vec(), vec()],
        out_specs=pl.BlockSpec((blk, D), lambda i: (i, 0)),
    )(p0, p1, nfeat, w1a, w1n, nW2, nb1, nb2, ng, nbt)


def kernel(efeat, nfeat, edge_index, eW1, eb1, eW2, eb2, eg, ebt,
           nW1, nb1, nW2, nb2, ng, nbt):
    src = edge_index[0]
    dst = edge_index[1]
    dst3 = dst.reshape(NW, NCHUNK, CH)
    w1e, w1s, w1d = eW1[0:D], eW1[D:2 * D], eW1[2 * D:3 * D]

    ns, nd = _project(nfeat, w1s, w1d)

    gs = []
    off = 0
    for u in UNITS:
        sz = u * UNIT
        sd = jnp.stack([src[off:off + sz].reshape(NW, u, CH),
                        dst[off:off + sz].reshape(NW, u, CH)], axis=2)
        gs.append(_gather_u(u)(sd, ns, nd))
        off += sz

    buf = None
    k0 = 0
    for u, g_k in zip(UNITS, gs):
        buf = _edge_mlp_chunk(k0, u, buf, efeat, g_k, w1e, eW2,
                              eb1.reshape(1, D), eb2.reshape(1, D),
                              eg.reshape(1, D), ebt.reshape(1, D))
        k0 += u
    efeat_new = buf
    aggp = _scatter(efeat_new, dst3)
    nfeat_new = _node_mlp(aggp[0], aggp[1], nfeat, nW1[0:D], nW1[D:2 * D],
                          nW2, nb1.reshape(1, D), nb2.reshape(1, D),
                          ng.reshape(1, D), nbt.reshape(1, D))
    return (efeat_new, nfeat_new)


# speedup vs baseline: 1.0508x; 1.0358x over previous
import functools

import jax
import jax.numpy as jnp
from jax import lax
from jax.experimental import pallas as pl
from jax.experimental.pallas import tpu as pltpu
from jax.experimental.pallas import tpu_sc as plsc

E = 320000
N = 10000
D = 128

NC = 2
NS = 16
NW = NC * NS
RPW = E // NW
CH = 80
NCHUNK = RPW // CH

UNIT = NW * CH
UNITS = (30, 30, 30, 29, 6)
CK = len(UNITS)
NBUF = 3

NIO = 10
RIO = N // NIO
ZB = 40

_SC_MESH = plsc.VectorSubcoreMesh(
    core_axis_name="c", subcore_axis_name="s", num_cores=NC, num_subcores=NS)


def _proj_body(nf_ref, ws_ref, wd_ref, ns_ref, nd_ref):
    x = nf_ref[...]
    ns_ref[...] = jnp.dot(x, ws_ref[...], preferred_element_type=jnp.float32)
    nd_ref[...] = jnp.dot(x, wd_ref[...], preferred_element_type=jnp.float32)


def _project(nfeat, w1s, w1d):
    blk = 2000
    return pl.pallas_call(
        _proj_body,
        out_shape=(jax.ShapeDtypeStruct((N, D), jnp.float32),
                   jax.ShapeDtypeStruct((N, D), jnp.float32)),
        grid=(N // blk,),
        in_specs=[pl.BlockSpec((blk, D), lambda i: (i, 0)),
                  pl.BlockSpec((D, D), lambda i: (0, 0)),
                  pl.BlockSpec((D, D), lambda i: (0, 0))],
        out_specs=(pl.BlockSpec((blk, D), lambda i: (i, 0)),
                   pl.BlockSpec((blk, D), lambda i: (i, 0))),
    )(nfeat, w1s, w1d)


def _make_gather_body(nchc):
    def _gather_body(sd_hbm, ns_hbm, nd_hbm, out_hbm,
                     sdidx, bufa, sem_a, sem_b, sem_o):
        c = lax.axis_index("c")
        s = lax.axis_index("s")
        wid = s * NC + c
        base = wid * (nchc * CH)
        pltpu.sync_copy(sd_hbm.at[wid], sdidx)

        pltpu.async_copy(ns_hbm.at[sdidx.at[0, 0]], bufa.at[0], sem_a.at[0])
        pltpu.async_copy(ns_hbm.at[sdidx.at[1, 0]], bufa.at[1], sem_a.at[1])

        @pl.loop(0, nchc)
        def chunk(j):
            slot = lax.rem(j, NBUF)

            pltpu.make_async_copy(ns_hbm.at[sdidx.at[j, 0]], bufa.at[slot],
                                  sem_a.at[slot]).wait()
            pltpu.async_copy(nd_hbm.at[sdidx.at[j, 1]], bufa.at[slot],
                             sem_b.at[slot], add=True)

            @pl.when(j + 2 < nchc)
            def _issue_next():
                nslot = lax.rem(j + 2, NBUF)

                @pl.when(j >= 1)
                def _drain_out():
                    pltpu.make_async_copy(
                        bufa.at[nslot],
                        out_hbm.at[pl.ds(base + (j - 1) * CH, CH)],
                        sem_o.at[nslot]).wait()

                pltpu.async_copy(ns_hbm.at[sdidx.at[j + 2, 0]],
                                 bufa.at[nslot], sem_a.at[nslot])

            pltpu.make_async_copy(nd_hbm.at[sdidx.at[j, 1]], bufa.at[slot],
                                  sem_b.at[slot]).wait()
            pltpu.async_copy(bufa.at[slot],
                             out_hbm.at[pl.ds(base + j * CH, CH)],
                             sem_o.at[slot])

        for tail in (nchc - 3, nchc - 2, nchc - 1):
            slot = tail % NBUF
            pltpu.make_async_copy(
                bufa.at[slot], out_hbm.at[pl.ds(base + tail * CH, CH)],
                sem_o.at[slot]).wait()

    return _gather_body


@functools.cache
def _gather_u(nchc):
    return pl.kernel(
        _make_gather_body(nchc),
        out_type=jax.ShapeDtypeStruct((nchc * UNIT, D), jnp.float32),
        mesh=_SC_MESH,
        scratch_types=[
            pltpu.VMEM((nchc, 2, CH), jnp.int32),
            pltpu.VMEM((NBUF, CH, D), jnp.float32),
            pltpu.SemaphoreType.DMA((NBUF,)),
            pltpu.SemaphoreType.DMA((NBUF,)),
            pltpu.SemaphoreType.DMA((NBUF,)),
        ],
    )


def _edge_body(_buf_ref, ef_ref, g_ref, w1_ref, w2_ref, b1_ref, b2_ref,
               gg_ref, bt_ref, out_ref):
    x = ef_ref[...]
    pre = (jnp.dot(x, w1_ref[...], preferred_element_type=jnp.float32)
           + g_ref[...] + b1_ref[...])
    h = pre * jax.nn.sigmoid(pre)
    y = jnp.dot(h, w2_ref[...], preferred_element_type=jnp.float32) + b2_ref[...]
    m = jnp.mean(y, axis=-1, keepdims=True)
    d = y - m
    v = jnp.mean(d * d, axis=-1, keepdims=True)
    out_ref[...] = d * lax.rsqrt(v + 1e-5) * gg_ref[...] + bt_ref[...] + x


_EBLK = UNIT


def _edge_body0(ef_ref, g_ref, w1_ref, w2_ref, b1_ref, b2_ref, gg_ref, bt_ref,
                out_ref):
    _edge_body(None, ef_ref, g_ref, w1_ref, w2_ref, b1_ref, b2_ref, gg_ref,
               bt_ref, out_ref)


def _edge_mlp_chunk(k0, u, buf, efeat, g_k, w1e, eW2, eb1, eb2, eg, ebt):
    vec = lambda: pl.BlockSpec((1, D), lambda i: (0, 0))
    row_specs = [
        pl.BlockSpec((_EBLK, D), lambda i, k0=k0: (k0 + i, 0)),
        pl.BlockSpec((_EBLK, D), lambda i: (i, 0)),
        pl.BlockSpec((D, D), lambda i: (0, 0)),
        pl.BlockSpec((D, D), lambda i: (0, 0)),
        vec(), vec(), vec(), vec(),
    ]
    args = (efeat, g_k, w1e, eW2, eb1, eb2, eg, ebt)
    if k0 == 0:
        return pl.pallas_call(
            _edge_body0,
            out_shape=jax.ShapeDtypeStruct((E, D), jnp.float32),
            grid=(u,),
            in_specs=row_specs,
            out_specs=pl.BlockSpec((_EBLK, D), lambda i: (i, 0)),
        )(*args)
    return pl.pallas_call(
        _edge_body,
        out_shape=jax.ShapeDtypeStruct((E, D), jnp.float32),
        grid=(u,),
        in_specs=[pl.BlockSpec(memory_space=pl.ANY)] + row_specs,
        out_specs=pl.BlockSpec((_EBLK, D), lambda i, k0=k0: (k0 + i, 0)),
        input_output_aliases={0: 0},
    )(buf, *args)


def _scatter_body(ef_hbm, dst_hbm, out_hbm, didx, rows, agg, sem_r, sem_w):
    c = lax.axis_index("c")
    s = lax.axis_index("s")
    wid = s * NC + c
    base = wid * RPW

    @pl.when(s < NIO)
    def _init():
        r0ref = rows.at[0]

        @pl.loop(0, ZB)
        def zrow(r):
            for cc in range(8):
                r0ref[r, pl.ds(cc * 16, 16)] = jnp.zeros((16,), jnp.float32)

        for k in range(RIO // ZB):
            pltpu.sync_copy(rows.at[0, pl.ds(0, ZB)],
                            agg.at[pl.ds(s * RIO + k * ZB, ZB)])

    plsc.subcore_barrier()

    pltpu.sync_copy(dst_hbm.at[wid], didx)
    pltpu.async_copy(ef_hbm.at[pl.ds(base, CH)], rows.at[0], sem_r.at[0])
    pltpu.async_copy(ef_hbm.at[pl.ds(base + CH, CH)], rows.at[1], sem_r.at[1])

    @pl.loop(0, NCHUNK)
    def chunk(j):
        slot = lax.rem(j, NBUF)

        pltpu.make_async_copy(ef_hbm.at[pl.ds(base + j * CH, CH)],
                              rows.at[slot], sem_r.at[slot]).wait()
        pltpu.async_copy(rows.at[slot], agg.at[didx.at[j]], sem_w.at[slot],
                         add=True)

        @pl.when(j + 2 < NCHUNK)
        def _issue_next():
            nslot = lax.rem(j + 2, NBUF)

            @pl.when(j >= 1)
            def _drain_prev():
                pltpu.make_async_copy(rows.at[nslot],
                                      agg.at[didx.at[j - 1]],
                                      sem_w.at[nslot]).wait()

            pltpu.async_copy(ef_hbm.at[pl.ds(base + (j + 2) * CH, CH)],
                             rows.at[nslot], sem_r.at[nslot])

    for tail in (NCHUNK - 3, NCHUNK - 2, NCHUNK - 1):
        slot = tail % NBUF
        pltpu.make_async_copy(rows.at[slot], agg.at[didx.at[tail]],
                              sem_w.at[slot]).wait()

    plsc.subcore_barrier()

    @pl.when(s < NIO)
    def _writeout():
        for k in range(RIO // ZB):
            r0 = s * RIO + k * ZB
            pltpu.sync_copy(agg.at[pl.ds(r0, ZB)], rows.at[0, pl.ds(0, ZB)])
            pltpu.sync_copy(rows.at[0, pl.ds(0, ZB)],
                            out_hbm.at[c, pl.ds(r0, ZB)])


_scatter = pl.kernel(
    _scatter_body,
    out_type=jax.ShapeDtypeStruct((NC, N, D), jnp.float32),
    mesh=_SC_MESH,
    scratch_types=[
        pltpu.VMEM((NCHUNK, CH), jnp.int32),
        pltpu.VMEM((NBUF, CH, D), jnp.float32),
        pltpu.VMEM_SHARED((N, D), jnp.float32),
        pltpu.SemaphoreType.DMA((NBUF,)),
        pltpu.SemaphoreType.DMA((NBUF,)),
    ],
)


def _node_body(p0_ref, p1_ref, nf_ref, w1a_ref, w1n_ref, w2_ref, b1_ref,
               b2_ref, gg_ref, bt_ref, out_ref):
    a = p0_ref[...] + p1_ref[...]
    x = nf_ref[...]
    pre = (jnp.dot(a, w1a_ref[...], preferred_element_type=jnp.float32)
           + jnp.dot(x, w1n_ref[...], preferred_element_type=jnp.float32)
           + b1_ref[...])
    h = pre * jax.nn.sigmoid(pre)
    y = jnp.dot(h, w2_ref[...], preferred_element_type=jnp.float32) + b2_ref[...]
    m = jnp.mean(y, axis=-1, keepdims=True)
    d = y - m
    v = jnp.mean(d * d, axis=-1, keepdims=True)
    out_ref[...] = d * lax.rsqrt(v + 1e-5) * gg_ref[...] + bt_ref[...] + x


def _node_mlp(p0, p1, nfeat, w1a, w1n, nW2, nb1, nb2, ng, nbt):
    blk = 2000
    vec = lambda: pl.BlockSpec((1, D), lambda i: (0, 0))
    mat = lambda: pl.BlockSpec((D, D), lambda i: (0, 0))
    return pl.pallas_call(
        _node_body,
        out_shape=jax.ShapeDtypeStruct((N, D), jnp.float32),
        grid=(N // blk,),
        in_specs=[pl.BlockSpec((blk, D), lambda i: (i, 0)),
                  pl.BlockSpec((blk, D), lambda i: (i, 0)),
                  pl.BlockSpec((blk, D), lambda i: (i, 0)),
                  mat(), mat(), mat(),
                  vec(), vec(), vec(), vec()],
        out_specs=pl.BlockSpec((blk, D), lambda i: (i, 0)),
    )(p0, p1, nfeat, w1a, w1n, nW2, nb1, nb2, ng, nbt)


def kernel(efeat, nfeat, edge_index, eW1, eb1, eW2, eb2, eg, ebt,
           nW1, nb1, nW2, nb2, ng, nbt):
    src = edge_index[0]
    dst = edge_index[1]
    dst3 = dst.reshape(NW, NCHUNK, CH)
    w1e, w1s, w1d = eW1[0:D], eW1[D:2 * D], eW1[2 * D:3 * D]

    ns, nd = _project(nfeat, w1s, w1d)

    gs = []
    off = 0
    for u in UNITS:
        sz = u * UNIT
        sd = jnp.stack([src[off:off + sz].reshape(NW, u, CH),
                        dst[off:off + sz].reshape(NW, u, CH)], axis=2)
        gs.append(_gather_u(u)(sd, ns, nd))
        off += sz

    buf = None
    k0 = 0
    for u, g_k in zip(UNITS, gs):
        buf = _edge_mlp_chunk(k0, u, buf, efeat, g_k, w1e, eW2,
                              eb1.reshape(1, D), eb2.reshape(1, D),
                              eg.reshape(1, D), ebt.reshape(1, D))
        k0 += u
    efeat_new = buf
    aggp = _scatter(efeat_new, dst3)
    nfeat_new = _node_mlp(aggp[0], aggp[1], nfeat, nW1[0:D], nW1[D:2 * D],
                          nW2, nb1.reshape(1, D), nb2.reshape(1, D),
                          ng.reshape(1, D), nbt.reshape(1, D))
    return (efeat_new, nfeat_new)
